# Initial kernel scaffold; baseline (speedup 1.0000x reference)
import functools

import jax
import jax.numpy as jnp
from jax import lax
from jax.experimental import pallas as pl
from jax.experimental.pallas import tpu as pltpu
from jax.experimental.pallas import tpu_sc as plsc

_N = 50000
_E = 800000
_DH = 80
_CH = 128
_NP = 50048
_EP = 802816
_NC = 2
_NS = 16
_ROWCHUNKS = _NP // _CH
_F32 = jnp.float32


def _sc_mesh():
    return plsc.VectorSubcoreMesh(core_axis_name="c", subcore_axis_name="s")


def _zero_acc(zv_hbm, zv, acc, s):
    pltpu.sync_copy(zv_hbm, zv)
    nz = (_ROWCHUNKS + _NS - 1) // _NS

    def zbody(j, carry):
        ch = s + j * _NS

        @pl.when(ch < _ROWCHUNKS)
        def _():
            pltpu.sync_copy(zv, acc.at[pl.ds(ch * _CH, _CH)])

        return carry

    lax.fori_loop(0, nz, zbody, 0)


def _copy_out(acc, out_view, s):
    nz = (_ROWCHUNKS + _NS - 1) // _NS

    def obody(j, carry):
        ch = s + j * _NS

        @pl.when(ch < _ROWCHUNKS)
        def _():
            pltpu.sync_copy(acc.at[pl.ds(ch * _CH, _CH)],
                            out_view.at[pl.ds(ch * _CH, _CH)])

        return carry

    lax.fori_loop(0, nz, obody, 0)


def _sc_edge_stats(dst_p, ea8, zeros8):
    def body(dst_hbm, ea_hbm, z_hbm, out_hbm, acc, idx_v, ea_v, zv):
        c = lax.axis_index("c")
        s = lax.axis_index("s")
        _zero_acc(z_hbm, zv, acc, s)
        plsc.subcore_barrier()

        per_tile = _EP // (_NC * _NS)
        tile_base = (c * _NS + s) * per_tile
        nch = per_tile // _CH

        def ebody(j, carry):
            base = tile_base + j * _CH
            pltpu.sync_copy(dst_hbm.at[pl.ds(base, _CH)], idx_v)
            pltpu.sync_copy(ea_hbm.at[pl.ds(base, _CH)], ea_v)
            pltpu.sync_copy(ea_v, acc.at[idx_v], add=True)
            return carry

        lax.fori_loop(0, nch, ebody, 0)
        plsc.subcore_barrier()
        _copy_out(acc, out_hbm.at[c], s)

    return pl.kernel(
        body,
        out_type=jax.ShapeDtypeStruct((_NC, _NP, 8), _F32),
        mesh=_sc_mesh(),
        scratch_types=[
            pltpu.MemorySpace.VMEM_SHARED((_NP, 8), _F32),
            pltpu.VMEM((_CH,), jnp.int32),
            pltpu.VMEM((_CH, 8), _F32),
            pltpu.VMEM((_CH, 8), _F32),
        ],
    )(dst_p, ea8, zeros8)


def _sc_gather_segsum(src_p, dst_p, hh2, zeros40):
    def body(src_hbm, dst_hbm, tab_hbm, z_hbm, out_hbm,
             acc, sidx_v, didx_v, rows_v, zv, sem):
        c = lax.axis_index("c")
        s = lax.axis_index("s")
        _zero_acc(z_hbm, zv, acc, s)
        plsc.subcore_barrier()

        per_tile = _EP // _NS
        tile_base = s * per_tile
        nch = per_tile // _CH

        def ebody(j, carry):
            base = tile_base + j * _CH
            pltpu.sync_copy(src_hbm.at[pl.ds(base, _CH)], sidx_v)
            pltpu.sync_copy(dst_hbm.at[pl.ds(base, _CH)], didx_v)
            pltpu.async_copy(tab_hbm.at[c].at[sidx_v], rows_v, sem).wait()
            pltpu.sync_copy(rows_v, acc.at[didx_v], add=True)
            return carry

        lax.fori_loop(0, nch, ebody, 0)
        plsc.subcore_barrier()
        _copy_out(acc, out_hbm.at[c], s)

    return pl.kernel(
        body,
        out_type=jax.ShapeDtypeStruct((_NC, _NP, 40), _F32),
        mesh=_sc_mesh(),
        scratch_types=[
            pltpu.MemorySpace.VMEM_SHARED((_NP, 40), _F32),
            pltpu.VMEM((_CH,), jnp.int32),
            pltpu.VMEM((_CH,), jnp.int32),
            pltpu.VMEM((_CH, 40), _F32),
            pltpu.VMEM((_CH, 40), _F32),
            pltpu.SemaphoreType.DMA,
        ],
    )(src_p, dst_p, hh2, zeros40)


def _tc_round(hh2, g2, s7p, At, Wt, Mt, bb):
    def body(hh_ref, g_ref, s_ref, At_ref, Wt_ref, Mt_ref, bb_ref, out_ref):
        hh = jnp.concatenate([hh_ref[0], hh_ref[1]], axis=1)
        g = jnp.concatenate([g_ref[0], g_ref[1]], axis=1)
        sv = s_ref[0] + s_ref[1]
        t = (jnp.dot(hh, At_ref[...], preferred_element_type=_F32)
             + jnp.dot(g, Wt_ref[...], preferred_element_type=_F32)
             + jnp.dot(sv, Mt_ref[...], preferred_element_type=_F32)
             + bb_ref[...])
        r = jnp.maximum(t, 0.0)
        out_ref[0] = r[:, :40]
        out_ref[1] = r[:, 40:]

    grid = (_ROWCHUNKS,)
    return pl.pallas_call(
        body,
        grid=grid,
        in_specs=[
            pl.BlockSpec((_NC, _CH, 40), lambda i: (0, i, 0)),
            pl.BlockSpec((_NC, _CH, 40), lambda i: (0, i, 0)),
            pl.BlockSpec((_NC, _CH, 8), lambda i: (0, i, 0)),
            pl.BlockSpec((_DH, _DH), lambda i: (0, 0)),
            pl.BlockSpec((_DH, _DH), lambda i: (0, 0)),
            pl.BlockSpec((8, _DH), lambda i: (0, 0)),
            pl.BlockSpec((1, _DH), lambda i: (0, 0)),
        ],
        out_specs=pl.BlockSpec((_NC, _CH, 40), lambda i: (0, i, 0)),
        out_shape=jax.ShapeDtypeStruct((_NC, _NP, 40), _F32),
        compiler_params=pltpu.CompilerParams(
            dimension_semantics=("arbitrary",),
        ),
    )(hh2, g2, s7p, At, Wt, Mt, bb)


def _tc_readout(hh2, h0p, Pt, Qt, Rb, out_wt, out_b2):
    def body(hh_ref, h0_ref, Pt_ref, Qt_ref, Rb_ref, ow_ref, ob_ref,
             out_ref, acc_ref):
        i = pl.program_id(0)

        @pl.when(i == 0)
        def _():
            acc_ref[...] = jnp.zeros_like(acc_ref)

        hh = jnp.concatenate([hh_ref[0], hh_ref[1]], axis=1)
        reads = (jnp.dot(hh, Pt_ref[...], preferred_element_type=_F32)
                 + jnp.dot(h0_ref[...], Qt_ref[...], preferred_element_type=_F32)
                 + Rb_ref[...])
        reads = jnp.maximum(reads, 0.0)
        gid = i * _CH + lax.broadcasted_iota(jnp.int32, (_CH, 1), 0)
        reads = jnp.where(gid < _N, reads, 0.0)
        acc_ref[...] += jnp.sum(reads.reshape(_CH // 8, 8, 128), axis=0)

        @pl.when(i == _ROWCHUNKS - 1)
        def _():
            tot = jnp.sum(acc_ref[...], axis=0, keepdims=True)
            v = jnp.tanh(tot)
            out_ref[...] = jnp.dot(v, ow_ref[...],
                                   preferred_element_type=_F32) + ob_ref[...]

    grid = (_ROWCHUNKS,)
    return pl.pallas_call(
        body,
        grid=grid,
        in_specs=[
            pl.BlockSpec((_NC, _CH, 40), lambda i: (0, i, 0)),
            pl.BlockSpec((_CH, _DH), lambda i: (i, 0)),
            pl.BlockSpec((_DH, 128), lambda i: (0, 0)),
            pl.BlockSpec((_DH, 128), lambda i: (0, 0)),
            pl.BlockSpec((1, 128), lambda i: (0, 0)),
            pl.BlockSpec((128, 1), lambda i: (0, 0)),
            pl.BlockSpec((1, 1), lambda i: (0, 0)),
        ],
        out_specs=pl.BlockSpec((1, 1), lambda i: (0, 0)),
        out_shape=jax.ShapeDtypeStruct((1, 1), _F32),
        scratch_shapes=[pltpu.VMEM((8, 128), _F32)],
        compiler_params=pltpu.CompilerParams(
            dimension_semantics=("arbitrary",),
        ),
    )(hh2, h0p, Pt, Qt, Rb, out_wt, out_b2)


def kernel(h, edge_index, edge_attr, R_w, R_b, E_w, E_b,
           U0_w, U0_b, U1_w, U1_b, U2_w, U2_b,
           V0_w, V0_b, V1_w, V1_b, V2_w, V2_b,
           out_w, out_b):
    src = edge_index[0]
    dst = edge_index[1]

    src_p = jnp.zeros((_EP,), jnp.int32).at[:_E].set(src)
    dst_p = jnp.full((_EP,), _N, jnp.int32).at[:_E].set(dst)
    ea8 = jnp.zeros((_EP, 8), _F32)
    ea8 = ea8.at[:_E, :6].set(edge_attr).at[:_E, 6].set(1.0)

    h0p = jnp.zeros((_NP, _DH), _F32).at[:_N, :75].set(h)
    hh2 = jnp.stack([h0p[:, :40], h0p[:, 40:]])

    zeros8 = jnp.zeros((_CH, 8), _F32)
    zeros40 = jnp.zeros((_CH, 40), _F32)

    Uw = [U0_w, U1_w, U2_w]
    Ub = [U0_b, U1_b, U2_b]
    Vw = [V0_w, V1_w, V2_w]
    Vb = [V0_b, V1_b, V2_b]
    Ats, Wts, Mts, bbs = [], [], [], []
    for k in range(3):
        A = Uw[k][:, :75]
        B = Uw[k][:, 75:150]
        C = Uw[k][:, 150:156]
        W = B @ Vw[k]
        At = jnp.zeros((_DH, _DH), _F32).at[:75, :75].set(A.T)
        Wt = jnp.zeros((_DH, _DH), _F32).at[:75, :75].set(W.T)
        m6 = C @ E_w
        v7 = C @ E_b + B @ Vb[k]
        Mt = jnp.zeros((8, _DH), _F32)
        Mt = Mt.at[:6, :75].set(m6.T).at[6, :75].set(v7)
        bb = jnp.zeros((1, _DH), _F32).at[0, :75].set(Ub[k])
        Ats.append(At); Wts.append(Wt); Mts.append(Mt); bbs.append(bb)

    Pt = jnp.zeros((_DH, 128), _F32).at[:75, :].set(R_w[:, :75].T)
    Qt = jnp.zeros((_DH, 128), _F32).at[:75, :].set(R_w[:, 75:].T)
    Rb2 = R_b.reshape(1, 128)
    out_wt = out_w.reshape(1, 128).T
    out_b2 = out_b.reshape(1, 1)

    s7p = _sc_edge_stats(dst_p, ea8, zeros8)

    for k in range(3):
        g2 = _sc_gather_segsum(src_p, dst_p, hh2, zeros40)
        hh2 = _tc_round(hh2, g2, s7p, Ats[k], Wts[k], Mts[k], bbs[k])

    return _tc_readout(hh2, h0p, Pt, Qt, Rb2, out_wt, out_b2)


# speedup vs baseline: 2.3497x; 2.3497x over previous
import functools

import jax
import jax.numpy as jnp
from jax import lax
from jax.experimental import pallas as pl
from jax.experimental.pallas import tpu as pltpu
from jax.experimental.pallas import tpu_sc as plsc

_N = 50000
_E = 800000
_DH = 80
_CH = 128
_NP = 50048
_EP = 802816
_NC = 2
_NS = 16
_ROWCHUNKS = _NP // _CH
_F32 = jnp.float32


def _sc_mesh():
    return plsc.VectorSubcoreMesh(core_axis_name="c", subcore_axis_name="s")


def _zero_acc(zv_hbm, acc, s):
    nz = (_ROWCHUNKS + _NS - 1) // _NS

    def zbody(j, carry):
        ch = s + j * _NS

        @pl.when(ch < _ROWCHUNKS)
        def _():
            pltpu.sync_copy(zv_hbm, acc.at[pl.ds(ch * _CH, _CH)])

        return carry

    lax.fori_loop(0, nz, zbody, 0)


def _copy_out(acc, out_view, s):
    nz = (_ROWCHUNKS + _NS - 1) // _NS

    def obody(j, carry):
        ch = s + j * _NS

        @pl.when(ch < _ROWCHUNKS)
        def _():
            pltpu.sync_copy(acc.at[pl.ds(ch * _CH, _CH)],
                            out_view.at[pl.ds(ch * _CH, _CH)])

        return carry

    lax.fori_loop(0, nz, obody, 0)


def _sc_edge_stats(dst_p, ea8, zeros8):
    def body(dst_hbm, ea_hbm, z_hbm, out_hbm, acc, idx_v, ea_v):
        c = lax.axis_index("c")
        s = lax.axis_index("s")
        _zero_acc(z_hbm, acc, s)
        plsc.subcore_barrier()

        per_tile = _EP // (_NC * _NS)
        tile_base = (c * _NS + s) * per_tile
        nch = per_tile // _CH

        def ebody(j, carry):
            base = tile_base + j * _CH
            pltpu.sync_copy(dst_hbm.at[pl.ds(base, _CH)], idx_v)
            pltpu.sync_copy(ea_hbm.at[pl.ds(base, _CH)], ea_v)
            pltpu.sync_copy(ea_v, acc.at[idx_v], add=True)
            return carry

        lax.fori_loop(0, nch, ebody, 0)
        plsc.subcore_barrier()
        _copy_out(acc, out_hbm.at[c], s)

    return pl.kernel(
        body,
        out_type=jax.ShapeDtypeStruct((_NC, _NP, 8), _F32),
        mesh=_sc_mesh(),
        scratch_types=[
            pltpu.MemorySpace.VMEM_SHARED((_NP, 8), _F32),
            pltpu.VMEM((_CH,), jnp.int32),
            pltpu.VMEM((_CH, 8), _F32),
        ],
        compiler_params=pltpu.CompilerParams(use_tc_tiling_on_sc=False),
    )(dst_p, ea8, zeros8)


def _sc_gather_segsum(src_p, dst_p, hh2, zeros40):
    def body(src_hbm, dst_hbm, tab_hbm, z_hbm, out_hbm,
             acc, sidx_v, didx_v, rows_v, sem):
        c = lax.axis_index("c")
        s = lax.axis_index("s")
        _zero_acc(z_hbm, acc, s)
        plsc.subcore_barrier()

        per_tile = _EP // _NS
        tile_base = s * per_tile
        nch = per_tile // _CH

        def ebody(j, carry):
            base = tile_base + j * _CH
            pltpu.sync_copy(src_hbm.at[pl.ds(base, _CH)], sidx_v)
            pltpu.sync_copy(dst_hbm.at[pl.ds(base, _CH)], didx_v)
            pltpu.async_copy(tab_hbm.at[c].at[sidx_v], rows_v, sem).wait()
            pltpu.sync_copy(rows_v, acc.at[didx_v], add=True)
            return carry

        lax.fori_loop(0, nch, ebody, 0)
        plsc.subcore_barrier()
        _copy_out(acc, out_hbm.at[c], s)

    return pl.kernel(
        body,
        out_type=jax.ShapeDtypeStruct((_NC, _NP, 40), _F32),
        mesh=_sc_mesh(),
        scratch_types=[
            pltpu.MemorySpace.VMEM_SHARED((_NP, 40), _F32),
            pltpu.VMEM((_CH,), jnp.int32),
            pltpu.VMEM((_CH,), jnp.int32),
            pltpu.VMEM((_CH, 40), _F32),
            pltpu.SemaphoreType.DMA,
        ],
        compiler_params=pltpu.CompilerParams(use_tc_tiling_on_sc=False),
    )(src_p, dst_p, hh2, zeros40)


def _tc_round(hh2, g2, s7p, At, Wt, Mt, bb):
    def body(hh_ref, g_ref, s_ref, At_ref, Wt_ref, Mt_ref, bb_ref, out_ref):
        hh = jnp.concatenate([hh_ref[0], hh_ref[1]], axis=1)
        g = jnp.concatenate([g_ref[0], g_ref[1]], axis=1)
        sv = s_ref[0] + s_ref[1]
        t = (jnp.dot(hh, At_ref[...], preferred_element_type=_F32)
             + jnp.dot(g, Wt_ref[...], preferred_element_type=_F32)
             + jnp.dot(sv, Mt_ref[...], preferred_element_type=_F32)
             + bb_ref[...])
        r = jnp.maximum(t, 0.0)
        out_ref[0] = r[:, :40]
        out_ref[1] = r[:, 40:]

    grid = (_ROWCHUNKS,)
    return pl.pallas_call(
        body,
        grid=grid,
        in_specs=[
            pl.BlockSpec((_NC, _CH, 40), lambda i: (0, i, 0)),
            pl.BlockSpec((_NC, _CH, 40), lambda i: (0, i, 0)),
            pl.BlockSpec((_NC, _CH, 8), lambda i: (0, i, 0)),
            pl.BlockSpec((_DH, _DH), lambda i: (0, 0)),
            pl.BlockSpec((_DH, _DH), lambda i: (0, 0)),
            pl.BlockSpec((8, _DH), lambda i: (0, 0)),
            pl.BlockSpec((1, _DH), lambda i: (0, 0)),
        ],
        out_specs=pl.BlockSpec((_NC, _CH, 40), lambda i: (0, i, 0)),
        out_shape=jax.ShapeDtypeStruct((_NC, _NP, 40), _F32),
        compiler_params=pltpu.CompilerParams(
            dimension_semantics=("arbitrary",),
        ),
    )(hh2, g2, s7p, At, Wt, Mt, bb)


def _tc_readout(hh2, h0p, Pt, Qt, Rb, out_wt, out_b2):
    def body(hh_ref, h0_ref, Pt_ref, Qt_ref, Rb_ref, ow_ref, ob_ref,
             out_ref, acc_ref):
        i = pl.program_id(0)

        @pl.when(i == 0)
        def _():
            acc_ref[...] = jnp.zeros_like(acc_ref)

        hh = jnp.concatenate([hh_ref[0], hh_ref[1]], axis=1)
        reads = (jnp.dot(hh, Pt_ref[...], preferred_element_type=_F32)
                 + jnp.dot(h0_ref[...], Qt_ref[...], preferred_element_type=_F32)
                 + Rb_ref[...])
        reads = jnp.maximum(reads, 0.0)
        gid = i * _CH + lax.broadcasted_iota(jnp.int32, (_CH, 1), 0)
        reads = jnp.where(gid < _N, reads, 0.0)
        acc_ref[...] += jnp.sum(reads.reshape(_CH // 8, 8, 128), axis=0)

        @pl.when(i == _ROWCHUNKS - 1)
        def _():
            tot = jnp.sum(acc_ref[...], axis=0, keepdims=True)
            v = jnp.tanh(tot)
            out_ref[...] = jnp.dot(v, ow_ref[...],
                                   preferred_element_type=_F32) + ob_ref[...]

    grid = (_ROWCHUNKS,)
    return pl.pallas_call(
        body,
        grid=grid,
        in_specs=[
            pl.BlockSpec((_NC, _CH, 40), lambda i: (0, i, 0)),
            pl.BlockSpec((_CH, _DH), lambda i: (i, 0)),
            pl.BlockSpec((_DH, 128), lambda i: (0, 0)),
            pl.BlockSpec((_DH, 128), lambda i: (0, 0)),
            pl.BlockSpec((1, 128), lambda i: (0, 0)),
            pl.BlockSpec((128, 1), lambda i: (0, 0)),
            pl.BlockSpec((1, 1), lambda i: (0, 0)),
        ],
        out_specs=pl.BlockSpec((1, 1), lambda i: (0, 0)),
        out_shape=jax.ShapeDtypeStruct((1, 1), _F32),
        scratch_shapes=[pltpu.VMEM((8, 128), _F32)],
        compiler_params=pltpu.CompilerParams(
            dimension_semantics=("arbitrary",),
        ),
    )(hh2, h0p, Pt, Qt, Rb, out_wt, out_b2)


def kernel(h, edge_index, edge_attr, R_w, R_b, E_w, E_b,
           U0_w, U0_b, U1_w, U1_b, U2_w, U2_b,
           V0_w, V0_b, V1_w, V1_b, V2_w, V2_b,
           out_w, out_b):
    src = edge_index[0]
    dst = edge_index[1]

    src_p = jnp.zeros((_EP,), jnp.int32).at[:_E].set(src)
    dst_p = jnp.full((_EP,), _N, jnp.int32).at[:_E].set(dst)
    ea8 = jnp.zeros((_EP, 8), _F32)
    ea8 = ea8.at[:_E, :6].set(edge_attr).at[:_E, 6].set(1.0)

    h0p = jnp.zeros((_NP, _DH), _F32).at[:_N, :75].set(h)
    hh2 = jnp.stack([h0p[:, :40], h0p[:, 40:]])

    zeros8 = jnp.zeros((_CH, 8), _F32)
    zeros40 = jnp.zeros((_CH, 40), _F32)

    Uw = [U0_w, U1_w, U2_w]
    Ub = [U0_b, U1_b, U2_b]
    Vw = [V0_w, V1_w, V2_w]
    Vb = [V0_b, V1_b, V2_b]
    Ats, Wts, Mts, bbs = [], [], [], []
    for k in range(3):
        A = Uw[k][:, :75]
        B = Uw[k][:, 75:150]
        C = Uw[k][:, 150:156]
        W = B @ Vw[k]
        At = jnp.zeros((_DH, _DH), _F32).at[:75, :75].set(A.T)
        Wt = jnp.zeros((_DH, _DH), _F32).at[:75, :75].set(W.T)
        m6 = C @ E_w
        v7 = C @ E_b + B @ Vb[k]
        Mt = jnp.zeros((8, _DH), _F32)
        Mt = Mt.at[:6, :75].set(m6.T).at[6, :75].set(v7)
        bb = jnp.zeros((1, _DH), _F32).at[0, :75].set(Ub[k])
        Ats.append(At); Wts.append(Wt); Mts.append(Mt); bbs.append(bb)

    Pt = jnp.zeros((_DH, 128), _F32).at[:75, :].set(R_w[:, :75].T)
    Qt = jnp.zeros((_DH, 128), _F32).at[:75, :].set(R_w[:, 75:].T)
    Rb2 = R_b.reshape(1, 128)
    out_wt = out_w.reshape(1, 128).T
    out_b2 = out_b.reshape(1, 1)

    s7p = _sc_edge_stats(dst_p, ea8, zeros8)

    for k in range(3):
        g2 = _sc_gather_segsum(src_p, dst_p, hh2, zeros40)
        hh2 = _tc_round(hh2, g2, s7p, Ats[k], Wts[k], Mts[k], bbs[k])

    return _tc_readout(hh2, h0p, Pt, Qt, Rb2, out_wt, out_b2)


# speedup vs baseline: 2.6097x; 1.1106x over previous
import functools

import jax
import jax.numpy as jnp
from jax import lax
from jax.experimental import pallas as pl
from jax.experimental.pallas import tpu as pltpu
from jax.experimental.pallas import tpu_sc as plsc

_N = 50000
_E = 800000
_DH = 80
_CH = 128
_NP = 50048
_EP = 802816
_NC = 2
_NS = 16
_ROWCHUNKS = _NP // _CH
_F32 = jnp.float32


def _sc_mesh():
    return plsc.VectorSubcoreMesh(core_axis_name="c", subcore_axis_name="s")


def _zero_acc(zv_hbm, acc, s):
    nz = (_ROWCHUNKS + _NS - 1) // _NS

    def zbody(j, carry):
        ch = s + j * _NS

        @pl.when(ch < _ROWCHUNKS)
        def _():
            pltpu.sync_copy(zv_hbm, acc.at[pl.ds(ch * _CH, _CH)])

        return carry

    lax.fori_loop(0, nz, zbody, 0)


def _copy_out(acc, out_view, s):
    nz = (_ROWCHUNKS + _NS - 1) // _NS

    def obody(j, carry):
        ch = s + j * _NS

        @pl.when(ch < _ROWCHUNKS)
        def _():
            pltpu.sync_copy(acc.at[pl.ds(ch * _CH, _CH)],
                            out_view.at[pl.ds(ch * _CH, _CH)])

        return carry

    lax.fori_loop(0, nz, obody, 0)


def _sc_edge_stats(dst_p, ea8, zeros8):
    def body(dst_hbm, ea_hbm, z_hbm, out_hbm, acc, idx_v, ea_v):
        c = lax.axis_index("c")
        s = lax.axis_index("s")
        _zero_acc(z_hbm, acc, s)
        plsc.subcore_barrier()

        per_tile = _EP // (_NC * _NS)
        tile_base = (c * _NS + s) * per_tile
        nch = per_tile // _CH

        def ebody(j, carry):
            base = tile_base + j * _CH
            pltpu.sync_copy(dst_hbm.at[pl.ds(base, _CH)], idx_v)
            pltpu.sync_copy(ea_hbm.at[pl.ds(base, _CH)], ea_v)
            pltpu.sync_copy(ea_v, acc.at[idx_v], add=True)
            return carry

        lax.fori_loop(0, nch, ebody, 0)
        plsc.subcore_barrier()
        _copy_out(acc, out_hbm.at[c], s)

    return pl.kernel(
        body,
        out_type=jax.ShapeDtypeStruct((_NC, _NP, 8), _F32),
        mesh=_sc_mesh(),
        scratch_types=[
            pltpu.MemorySpace.VMEM_SHARED((_NP, 8), _F32),
            pltpu.VMEM((_CH,), jnp.int32),
            pltpu.VMEM((_CH, 8), _F32),
        ],
        compiler_params=pltpu.CompilerParams(use_tc_tiling_on_sc=False),
    )(dst_p, ea8, zeros8)


_EC = 64


def _sc_gather_segsum(src2, dst2, hh2, zeros40):
    def body(src_hbm, dst_hbm, tab_hbm, z_hbm, out_hbm,
             acc, ss0, ss1, sd0, sd1, rows,
             gs0, gs1, is0, is1, ds0, ds1):
        c = lax.axis_index("c")
        s = lax.axis_index("s")
        gsem = [gs0, gs1]
        isem = [is0, is1]
        dsem = [ds0, ds1]
        ssidx = [ss0, ss1]
        sdidx = [sd0, sd1]
        _zero_acc(z_hbm, acc, s)
        plsc.subcore_barrier()

        per_tile = _EP // _NS
        nch = per_tile // _EC
        crow0 = s * nch

        def sidx_start(j, b):
            pltpu.make_async_copy(src_hbm.at[crow0 + j], ssidx[b],
                                  isem[b]).start()

        def sidx_wait(j, b):
            pltpu.make_async_copy(src_hbm.at[crow0 + j], ssidx[b],
                                  isem[b]).wait()

        def didx_start(j, b):
            pltpu.make_async_copy(dst_hbm.at[crow0 + j], sdidx[b],
                                  dsem[b]).start()

        def didx_wait(j, b):
            pltpu.make_async_copy(dst_hbm.at[crow0 + j], sdidx[b],
                                  dsem[b]).wait()

        def gather_start(b):
            pltpu.make_async_copy(tab_hbm.at[c].at[ssidx[b]], rows.at[b],
                                  gsem[b]).start()

        def gather_wait(b):
            pltpu.make_async_copy(tab_hbm.at[c].at[ssidx[b]], rows.at[b],
                                  gsem[b]).wait()

        for b in (0, 1):
            sidx_start(b, b)
            didx_start(b, b)
        for b in (0, 1):
            sidx_wait(b, b)
            gather_start(b)

        def mbody(jj, carry):
            for b in range(2):
                j = 2 * jj + b
                gather_wait(b)
                didx_wait(j, b)
                pltpu.sync_copy(rows.at[b], acc.at[sdidx[b]], add=True)

                @pl.when(j + 2 < nch)
                def _():
                    didx_start(j + 2, b)
                    sidx_start(j + 2, b)
                    sidx_wait(j + 2, b)
                    gather_start(b)

            return carry

        lax.fori_loop(0, nch // 2, mbody, 0)
        plsc.subcore_barrier()
        _copy_out(acc, out_hbm.at[c], s)

    return pl.kernel(
        body,
        out_type=jax.ShapeDtypeStruct((_NC, _NP, 40), _F32),
        mesh=_sc_mesh(),
        scratch_types=[
            pltpu.MemorySpace.VMEM_SHARED((_NP, 40), _F32),
            pltpu.VMEM((_EC,), jnp.int32),
            pltpu.VMEM((_EC,), jnp.int32),
            pltpu.VMEM((_EC,), jnp.int32),
            pltpu.VMEM((_EC,), jnp.int32),
            pltpu.VMEM((2, _EC, 40), _F32),
            pltpu.SemaphoreType.DMA,
            pltpu.SemaphoreType.DMA,
            pltpu.SemaphoreType.DMA,
            pltpu.SemaphoreType.DMA,
            pltpu.SemaphoreType.DMA,
            pltpu.SemaphoreType.DMA,
        ],
        compiler_params=pltpu.CompilerParams(use_tc_tiling_on_sc=False),
    )(src2, dst2, hh2, zeros40)


def _tc_round(hh2, g2, s7p, At, Wt, Mt, bb):
    def body(hh_ref, g_ref, s_ref, At_ref, Wt_ref, Mt_ref, bb_ref, out_ref):
        hh = jnp.concatenate([hh_ref[0], hh_ref[1]], axis=1)
        g = jnp.concatenate([g_ref[0], g_ref[1]], axis=1)
        sv = s_ref[0] + s_ref[1]
        t = (jnp.dot(hh, At_ref[...], preferred_element_type=_F32)
             + jnp.dot(g, Wt_ref[...], preferred_element_type=_F32)
             + jnp.dot(sv, Mt_ref[...], preferred_element_type=_F32)
             + bb_ref[...])
        r = jnp.maximum(t, 0.0)
        out_ref[0] = r[:, :40]
        out_ref[1] = r[:, 40:]

    grid = (_ROWCHUNKS,)
    return pl.pallas_call(
        body,
        grid=grid,
        in_specs=[
            pl.BlockSpec((_NC, _CH, 40), lambda i: (0, i, 0)),
            pl.BlockSpec((_NC, _CH, 40), lambda i: (0, i, 0)),
            pl.BlockSpec((_NC, _CH, 8), lambda i: (0, i, 0)),
            pl.BlockSpec((_DH, _DH), lambda i: (0, 0)),
            pl.BlockSpec((_DH, _DH), lambda i: (0, 0)),
            pl.BlockSpec((8, _DH), lambda i: (0, 0)),
            pl.BlockSpec((1, _DH), lambda i: (0, 0)),
        ],
        out_specs=pl.BlockSpec((_NC, _CH, 40), lambda i: (0, i, 0)),
        out_shape=jax.ShapeDtypeStruct((_NC, _NP, 40), _F32),
        compiler_params=pltpu.CompilerParams(
            dimension_semantics=("arbitrary",),
        ),
    )(hh2, g2, s7p, At, Wt, Mt, bb)


def _tc_readout(hh2, h0p, Pt, Qt, Rb, out_wt, out_b2):
    def body(hh_ref, h0_ref, Pt_ref, Qt_ref, Rb_ref, ow_ref, ob_ref,
             out_ref, acc_ref):
        i = pl.program_id(0)

        @pl.when(i == 0)
        def _():
            acc_ref[...] = jnp.zeros_like(acc_ref)

        hh = jnp.concatenate([hh_ref[0], hh_ref[1]], axis=1)
        reads = (jnp.dot(hh, Pt_ref[...], preferred_element_type=_F32)
                 + jnp.dot(h0_ref[...], Qt_ref[...], preferred_element_type=_F32)
                 + Rb_ref[...])
        reads = jnp.maximum(reads, 0.0)
        gid = i * _CH + lax.broadcasted_iota(jnp.int32, (_CH, 1), 0)
        reads = jnp.where(gid < _N, reads, 0.0)
        acc_ref[...] += jnp.sum(reads.reshape(_CH // 8, 8, 128), axis=0)

        @pl.when(i == _ROWCHUNKS - 1)
        def _():
            tot = jnp.sum(acc_ref[...], axis=0, keepdims=True)
            v = jnp.tanh(tot)
            out_ref[...] = jnp.dot(v, ow_ref[...],
                                   preferred_element_type=_F32) + ob_ref[...]

    grid = (_ROWCHUNKS,)
    return pl.pallas_call(
        body,
        grid=grid,
        in_specs=[
            pl.BlockSpec((_NC, _CH, 40), lambda i: (0, i, 0)),
            pl.BlockSpec((_CH, _DH), lambda i: (i, 0)),
            pl.BlockSpec((_DH, 128), lambda i: (0, 0)),
            pl.BlockSpec((_DH, 128), lambda i: (0, 0)),
            pl.BlockSpec((1, 128), lambda i: (0, 0)),
            pl.BlockSpec((128, 1), lambda i: (0, 0)),
            pl.BlockSpec((1, 1), lambda i: (0, 0)),
        ],
        out_specs=pl.BlockSpec((1, 1), lambda i: (0, 0)),
        out_shape=jax.ShapeDtypeStruct((1, 1), _F32),
        scratch_shapes=[pltpu.VMEM((8, 128), _F32)],
        compiler_params=pltpu.CompilerParams(
            dimension_semantics=("arbitrary",),
        ),
    )(hh2, h0p, Pt, Qt, Rb, out_wt, out_b2)


def kernel(h, edge_index, edge_attr, R_w, R_b, E_w, E_b,
           U0_w, U0_b, U1_w, U1_b, U2_w, U2_b,
           V0_w, V0_b, V1_w, V1_b, V2_w, V2_b,
           out_w, out_b):
    src = edge_index[0]
    dst = edge_index[1]

    src_p = jnp.zeros((_EP,), jnp.int32).at[:_E].set(src)
    dst_p = jnp.full((_EP,), _N, jnp.int32).at[:_E].set(dst)
    ea8 = jnp.zeros((_EP, 8), _F32)
    ea8 = ea8.at[:_E, :6].set(edge_attr).at[:_E, 6].set(1.0)

    h0p = jnp.zeros((_NP, _DH), _F32).at[:_N, :75].set(h)
    hh2 = jnp.stack([h0p[:, :40], h0p[:, 40:]])

    zeros8 = jnp.zeros((_CH, 8), _F32)
    zeros40 = jnp.zeros((_CH, 40), _F32)

    Uw = [U0_w, U1_w, U2_w]
    Ub = [U0_b, U1_b, U2_b]
    Vw = [V0_w, V1_w, V2_w]
    Vb = [V0_b, V1_b, V2_b]
    Ats, Wts, Mts, bbs = [], [], [], []
    for k in range(3):
        A = Uw[k][:, :75]
        B = Uw[k][:, 75:150]
        C = Uw[k][:, 150:156]
        W = B @ Vw[k]
        At = jnp.zeros((_DH, _DH), _F32).at[:75, :75].set(A.T)
        Wt = jnp.zeros((_DH, _DH), _F32).at[:75, :75].set(W.T)
        m6 = C @ E_w
        v7 = C @ E_b + B @ Vb[k]
        Mt = jnp.zeros((8, _DH), _F32)
        Mt = Mt.at[:6, :75].set(m6.T).at[6, :75].set(v7)
        bb = jnp.zeros((1, _DH), _F32).at[0, :75].set(Ub[k])
        Ats.append(At); Wts.append(Wt); Mts.append(Mt); bbs.append(bb)

    Pt = jnp.zeros((_DH, 128), _F32).at[:75, :].set(R_w[:, :75].T)
    Qt = jnp.zeros((_DH, 128), _F32).at[:75, :].set(R_w[:, 75:].T)
    Rb2 = R_b.reshape(1, 128)
    out_wt = out_w.reshape(1, 128).T
    out_b2 = out_b.reshape(1, 1)

    s7p = _sc_edge_stats(dst_p, ea8, zeros8)

    src2 = src_p.reshape(_EP // _EC, _EC)
    dst2 = dst_p.reshape(_EP // _EC, _EC)
    for k in range(3):
        g2 = _sc_gather_segsum(src2, dst2, hh2, zeros40)
        hh2 = _tc_round(hh2, g2, s7p, Ats[k], Wts[k], Mts[k], bbs[k])

    return _tc_readout(hh2, h0p, Pt, Qt, Rb2, out_wt, out_b2)


# speedup vs baseline: 3.5041x; 1.3427x over previous
import jax
import jax.numpy as jnp
from jax import lax
from jax.experimental import pallas as pl
from jax.experimental.pallas import tpu as pltpu
from jax.experimental.pallas import tpu_sc as plsc

_N = 50000
_E = 800000
_DH = 80
_CH = 128
_EC = 64
_NP = 50176
_EP = 802816
_NC = 2
_NS = 16
_ROWCHUNKS = _NP // _CH
_BN = 1024
_BE = 4096
_F32 = jnp.float32


def _sc_mesh():
    return plsc.VectorSubcoreMesh(core_axis_name="c", subcore_axis_name="s")


def _zero_acc(zv_hbm, acc, s):
    nz = (_ROWCHUNKS + _NS - 1) // _NS

    def zbody(j, carry):
        ch = s + j * _NS

        @pl.when(ch < _ROWCHUNKS)
        def _():
            pltpu.sync_copy(zv_hbm, acc.at[pl.ds(ch * _CH, _CH)])

        return carry

    lax.fori_loop(0, nz, zbody, 0)


def _copy_out(acc, out_view, s):
    nz = (_ROWCHUNKS + _NS - 1) // _NS

    def obody(j, carry):
        ch = s + j * _NS

        @pl.when(ch < _ROWCHUNKS)
        def _():
            pltpu.sync_copy(acc.at[pl.ds(ch * _CH, _CH)],
                            out_view.at[pl.ds(ch * _CH, _CH)])

        return carry

    lax.fori_loop(0, nz, obody, 0)


def _tc_edge_prep(edge_index, edge_attr):
    def body(ei_ref, ea_ref, src_ref, dst_ref, ea8_ref):
        i = pl.program_id(0)
        gid2 = i * _BE + lax.broadcasted_iota(jnp.int32, (_BE, 1), 0)
        m2 = gid2 < _E
        src_ref[...] = jnp.where(m2[:, 0], ei_ref[0], 0)
        dst_ref[...] = jnp.where(m2[:, 0], ei_ref[1], _N)
        full = jnp.concatenate(
            [ea_ref[...], jnp.ones((_BE, 1), _F32), jnp.zeros((_BE, 1), _F32)],
            axis=1)
        ea8_ref[...] = jnp.where(m2, full, 0.0)

    return pl.pallas_call(
        body,
        grid=(_EP // _BE,),
        in_specs=[
            pl.BlockSpec((2, _BE), lambda i: (0, i)),
            pl.BlockSpec((_BE, 6), lambda i: (i, 0)),
        ],
        out_specs=[
            pl.BlockSpec((_BE,), lambda i: (i,)),
            pl.BlockSpec((_BE,), lambda i: (i,)),
            pl.BlockSpec((_BE, 8), lambda i: (i, 0)),
        ],
        out_shape=[
            jax.ShapeDtypeStruct((_EP,), jnp.int32),
            jax.ShapeDtypeStruct((_EP,), jnp.int32),
            jax.ShapeDtypeStruct((_EP, 8), _F32),
        ],
        compiler_params=pltpu.CompilerParams(
            dimension_semantics=("arbitrary",),
        ),
    )(edge_index, edge_attr)


def _tc_node_prep(h):
    def body(h_ref, h0p_ref, hh2_ref):
        i = pl.program_id(0)
        row = i * _BN + lax.broadcasted_iota(jnp.int32, (_BN, 1), 0)
        hp = jnp.concatenate([h_ref[...], jnp.zeros((_BN, 5), _F32)], axis=1)
        hp = jnp.where(row < _N, hp, 0.0)
        h0p_ref[...] = hp
        hh2_ref[0] = hp[:, :40]
        hh2_ref[1] = hp[:, 40:]

    return pl.pallas_call(
        body,
        grid=(_NP // _BN,),
        in_specs=[pl.BlockSpec((_BN, 75), lambda i: (i, 0))],
        out_specs=[
            pl.BlockSpec((_BN, _DH), lambda i: (i, 0)),
            pl.BlockSpec((_NC, _BN, 40), lambda i: (0, i, 0)),
        ],
        out_shape=[
            jax.ShapeDtypeStruct((_NP, _DH), _F32),
            jax.ShapeDtypeStruct((_NC, _NP, 40), _F32),
        ],
        compiler_params=pltpu.CompilerParams(
            dimension_semantics=("arbitrary",),
        ),
    )(h)


def _sc_edge_stats(dst_p, ea8, zeros8):
    def body(dst_hbm, ea_hbm, z_hbm, out_hbm, acc, idx_v, ea_v):
        c = lax.axis_index("c")
        s = lax.axis_index("s")
        _zero_acc(z_hbm, acc, s)
        plsc.subcore_barrier()

        per_tile = _EP // (_NC * _NS)
        tile_base = (c * _NS + s) * per_tile
        nch = per_tile // _CH

        def ebody(j, carry):
            base = tile_base + j * _CH
            pltpu.sync_copy(dst_hbm.at[pl.ds(base, _CH)], idx_v)
            pltpu.sync_copy(ea_hbm.at[pl.ds(base, _CH)], ea_v)
            pltpu.sync_copy(ea_v, acc.at[idx_v], add=True)
            return carry

        lax.fori_loop(0, nch, ebody, 0)
        plsc.subcore_barrier()
        _copy_out(acc, out_hbm.at[c], s)

    return pl.kernel(
        body,
        out_type=jax.ShapeDtypeStruct((_NC, _NP, 8), _F32),
        mesh=_sc_mesh(),
        scratch_types=[
            pltpu.MemorySpace.VMEM_SHARED((_NP, 8), _F32),
            pltpu.VMEM((_CH,), jnp.int32),
            pltpu.VMEM((_CH, 8), _F32),
        ],
        compiler_params=pltpu.CompilerParams(use_tc_tiling_on_sc=False),
    )(dst_p, ea8, zeros8)


def _sc_gather_segsum(src_p, dst_p, hh2, zeros40):
    def body(src_hbm, dst_hbm, tab_hbm, z_hbm, out_hbm,
             acc, ss0, ss1, sd0, sd1, rows,
             gs0, gs1, is0, is1, ds0, ds1):
        c = lax.axis_index("c")
        s = lax.axis_index("s")
        gsem = [gs0, gs1]
        isem = [is0, is1]
        dsem = [ds0, ds1]
        ssidx = [ss0, ss1]
        sdidx = [sd0, sd1]
        _zero_acc(z_hbm, acc, s)
        plsc.subcore_barrier()

        per_tile = _EP // _NS
        nch = per_tile // _EC
        ebase = s * per_tile

        def sidx_start(j, b):
            pltpu.make_async_copy(src_hbm.at[pl.ds(ebase + j * _EC, _EC)],
                                  ssidx[b], isem[b]).start()

        def sidx_wait(j, b):
            pltpu.make_async_copy(src_hbm.at[pl.ds(ebase + j * _EC, _EC)],
                                  ssidx[b], isem[b]).wait()

        def didx_start(j, b):
            pltpu.make_async_copy(dst_hbm.at[pl.ds(ebase + j * _EC, _EC)],
                                  sdidx[b], dsem[b]).start()

        def didx_wait(j, b):
            pltpu.make_async_copy(dst_hbm.at[pl.ds(ebase + j * _EC, _EC)],
                                  sdidx[b], dsem[b]).wait()

        def gather_start(b):
            pltpu.make_async_copy(tab_hbm.at[c].at[ssidx[b]], rows.at[b],
                                  gsem[b]).start()

        def gather_wait(b):
            pltpu.make_async_copy(tab_hbm.at[c].at[ssidx[b]], rows.at[b],
                                  gsem[b]).wait()

        for b in (0, 1):
            sidx_start(b, b)
            didx_start(b, b)
        for b in (0, 1):
            sidx_wait(b, b)
            gather_start(b)

        def mbody(jj, carry):
            for b in range(2):
                j = 2 * jj + b
                gather_wait(b)
                didx_wait(j, b)
                pltpu.sync_copy(rows.at[b], acc.at[sdidx[b]], add=True)

                @pl.when(j + 2 < nch)
                def _():
                    didx_start(j + 2, b)
                    sidx_start(j + 2, b)
                    sidx_wait(j + 2, b)
                    gather_start(b)

            return carry

        lax.fori_loop(0, nch // 2, mbody, 0)
        plsc.subcore_barrier()
        _copy_out(acc, out_hbm.at[c], s)

    return pl.kernel(
        body,
        out_type=jax.ShapeDtypeStruct((_NC, _NP, 40), _F32),
        mesh=_sc_mesh(),
        scratch_types=[
            pltpu.MemorySpace.VMEM_SHARED((_NP, 40), _F32),
            pltpu.VMEM((_EC,), jnp.int32),
            pltpu.VMEM((_EC,), jnp.int32),
            pltpu.VMEM((_EC,), jnp.int32),
            pltpu.VMEM((_EC,), jnp.int32),
            pltpu.VMEM((2, _EC, 40), _F32),
            pltpu.SemaphoreType.DMA,
            pltpu.SemaphoreType.DMA,
            pltpu.SemaphoreType.DMA,
            pltpu.SemaphoreType.DMA,
            pltpu.SemaphoreType.DMA,
            pltpu.SemaphoreType.DMA,
        ],
        compiler_params=pltpu.CompilerParams(use_tc_tiling_on_sc=False),
    )(src_p, dst_p, hh2, zeros40)


def _tc_round(hh2, g2, s7p, At, Wt, Mt, bb):
    def body(hh_ref, g_ref, s_ref, At_ref, Wt_ref, Mt_ref, bb_ref, out_ref):
        hh = jnp.concatenate([hh_ref[0], hh_ref[1]], axis=1)
        g = jnp.concatenate([g_ref[0], g_ref[1]], axis=1)
        sv = s_ref[0] + s_ref[1]
        t = (jnp.dot(hh, At_ref[...], preferred_element_type=_F32)
             + jnp.dot(g, Wt_ref[...], preferred_element_type=_F32)
             + jnp.dot(sv, Mt_ref[...], preferred_element_type=_F32)
             + bb_ref[...])
        r = jnp.maximum(t, 0.0)
        out_ref[0] = r[:, :40]
        out_ref[1] = r[:, 40:]

    return pl.pallas_call(
        body,
        grid=(_NP // _BN,),
        in_specs=[
            pl.BlockSpec((_NC, _BN, 40), lambda i: (0, i, 0)),
            pl.BlockSpec((_NC, _BN, 40), lambda i: (0, i, 0)),
            pl.BlockSpec((_NC, _BN, 8), lambda i: (0, i, 0)),
            pl.BlockSpec((_DH, _DH), lambda i: (0, 0)),
            pl.BlockSpec((_DH, _DH), lambda i: (0, 0)),
            pl.BlockSpec((8, _DH), lambda i: (0, 0)),
            pl.BlockSpec((1, _DH), lambda i: (0, 0)),
        ],
        out_specs=pl.BlockSpec((_NC, _BN, 40), lambda i: (0, i, 0)),
        out_shape=jax.ShapeDtypeStruct((_NC, _NP, 40), _F32),
        compiler_params=pltpu.CompilerParams(
            dimension_semantics=("arbitrary",),
        ),
    )(hh2, g2, s7p, At, Wt, Mt, bb)


def _tc_readout(hh2, h0p, Pt, Qt, Rb, out_wt, out_b2):
    def body(hh_ref, h0_ref, Pt_ref, Qt_ref, Rb_ref, ow_ref, ob_ref,
             out_ref, acc_ref):
        i = pl.program_id(0)

        @pl.when(i == 0)
        def _():
            acc_ref[...] = jnp.zeros_like(acc_ref)

        hh = jnp.concatenate([hh_ref[0], hh_ref[1]], axis=1)
        reads = (jnp.dot(hh, Pt_ref[...], preferred_element_type=_F32)
                 + jnp.dot(h0_ref[...], Qt_ref[...], preferred_element_type=_F32)
                 + Rb_ref[...])
        reads = jnp.maximum(reads, 0.0)
        gid = i * _BN + lax.broadcasted_iota(jnp.int32, (_BN, 1), 0)
        reads = jnp.where(gid < _N, reads, 0.0)
        acc_ref[...] += jnp.sum(reads.reshape(_BN // 8, 8, 128), axis=0)

        @pl.when(i == _NP // _BN - 1)
        def _():
            tot = jnp.sum(acc_ref[...], axis=0, keepdims=True)
            v = jnp.tanh(tot)
            out_ref[...] = jnp.dot(v, ow_ref[...],
                                   preferred_element_type=_F32) + ob_ref[...]

    return pl.pallas_call(
        body,
        grid=(_NP // _BN,),
        in_specs=[
            pl.BlockSpec((_NC, _BN, 40), lambda i: (0, i, 0)),
            pl.BlockSpec((_BN, _DH), lambda i: (i, 0)),
            pl.BlockSpec((_DH, 128), lambda i: (0, 0)),
            pl.BlockSpec((_DH, 128), lambda i: (0, 0)),
            pl.BlockSpec((1, 128), lambda i: (0, 0)),
            pl.BlockSpec((128, 1), lambda i: (0, 0)),
            pl.BlockSpec((1, 1), lambda i: (0, 0)),
        ],
        out_specs=pl.BlockSpec((1, 1), lambda i: (0, 0)),
        out_shape=jax.ShapeDtypeStruct((1, 1), _F32),
        scratch_shapes=[pltpu.VMEM((8, 128), _F32)],
        compiler_params=pltpu.CompilerParams(
            dimension_semantics=("arbitrary",),
        ),
    )(hh2, h0p, Pt, Qt, Rb, out_wt, out_b2)


def kernel(h, edge_index, edge_attr, R_w, R_b, E_w, E_b,
           U0_w, U0_b, U1_w, U1_b, U2_w, U2_b,
           V0_w, V0_b, V1_w, V1_b, V2_w, V2_b,
           out_w, out_b):
    src_p, dst_p, ea8 = _tc_edge_prep(edge_index, edge_attr)
    h0p, hh2 = _tc_node_prep(h)

    zeros8 = jnp.zeros((_CH, 8), _F32)
    zeros40 = jnp.zeros((_CH, 40), _F32)

    Uw = [U0_w, U1_w, U2_w]
    Ub = [U0_b, U1_b, U2_b]
    Vw = [V0_w, V1_w, V2_w]
    Vb = [V0_b, V1_b, V2_b]
    Ats, Wts, Mts, bbs = [], [], [], []
    for k in range(3):
        A = Uw[k][:, :75]
        B = Uw[k][:, 75:150]
        C = Uw[k][:, 150:156]
        W = B @ Vw[k]
        At = jnp.zeros((_DH, _DH), _F32).at[:75, :75].set(A.T)
        Wt = jnp.zeros((_DH, _DH), _F32).at[:75, :75].set(W.T)
        m6 = C @ E_w
        v7 = C @ E_b + B @ Vb[k]
        Mt = jnp.zeros((8, _DH), _F32)
        Mt = Mt.at[:6, :75].set(m6.T).at[6, :75].set(v7)
        bb = jnp.zeros((1, _DH), _F32).at[0, :75].set(Ub[k])
        Ats.append(At); Wts.append(Wt); Mts.append(Mt); bbs.append(bb)

    Pt = jnp.zeros((_DH, 128), _F32).at[:75, :].set(R_w[:, :75].T)
    Qt = jnp.zeros((_DH, 128), _F32).at[:75, :].set(R_w[:, 75:].T)
    Rb2 = R_b.reshape(1, 128)
    out_wt = out_w.reshape(1, 128).T
    out_b2 = out_b.reshape(1, 1)

    s7p = _sc_edge_stats(dst_p, ea8, zeros8)

    for k in range(3):
        g2 = _sc_gather_segsum(src_p, dst_p, hh2, zeros40)
        hh2 = _tc_round(hh2, g2, s7p, Ats[k], Wts[k], Mts[k], bbs[k])

    return _tc_readout(hh2, h0p, Pt, Qt, Rb2, out_wt, out_b2)


# speedup vs baseline: 3.5991x; 1.0271x over previous
import jax
import jax.numpy as jnp
from jax import lax
from jax.experimental import pallas as pl
from jax.experimental.pallas import tpu as pltpu
from jax.experimental.pallas import tpu_sc as plsc

_N = 50000
_E = 800000
_DH = 80
_CH = 128
_EC = 64
_NP = 50176
_EP = 802816
_NC = 2
_NS = 16
_ROWCHUNKS = _NP // _CH
_BN = 1024
_BE = 4096
_F32 = jnp.float32


def _sc_mesh():
    return plsc.VectorSubcoreMesh(core_axis_name="c", subcore_axis_name="s")


def _zero_acc(zv_hbm, acc, s):
    nz = (_ROWCHUNKS + _NS - 1) // _NS

    def zbody(j, carry):
        ch = s + j * _NS

        @pl.when(ch < _ROWCHUNKS)
        def _():
            pltpu.sync_copy(zv_hbm, acc.at[pl.ds(ch * _CH, _CH)])

        return carry

    lax.fori_loop(0, nz, zbody, 0)


def _copy_out(acc, out_view, s):
    nz = (_ROWCHUNKS + _NS - 1) // _NS

    def obody(j, carry):
        ch = s + j * _NS

        @pl.when(ch < _ROWCHUNKS)
        def _():
            pltpu.sync_copy(acc.at[pl.ds(ch * _CH, _CH)],
                            out_view.at[pl.ds(ch * _CH, _CH)])

        return carry

    lax.fori_loop(0, nz, obody, 0)


def _tc_idx_prep(edge_index):
    def body(ei_ref, src_ref, dst_ref):
        i = pl.program_id(0)
        gid = i * _BE + lax.broadcasted_iota(jnp.int32, (_BE, 1), 0)
        m = (gid < _E)[:, 0]
        src_ref[...] = jnp.where(m, ei_ref[0], 0)
        dst_ref[...] = jnp.where(m, ei_ref[1], _N)

    return pl.pallas_call(
        body,
        grid=(_EP // _BE,),
        in_specs=[pl.BlockSpec((2, _BE), lambda i: (0, i))],
        out_specs=[
            pl.BlockSpec((_BE,), lambda i: (i,)),
            pl.BlockSpec((_BE,), lambda i: (i,)),
        ],
        out_shape=[
            jax.ShapeDtypeStruct((_EP,), jnp.int32),
            jax.ShapeDtypeStruct((_EP,), jnp.int32),
        ],
        compiler_params=pltpu.CompilerParams(
            dimension_semantics=("arbitrary",),
        ),
    )(edge_index)


def _tc_ea_prep(edge_attr):
    def body(ea_ref, ea8_ref):
        ea8_ref[:, 0:6] = ea_ref[...]
        ea8_ref[:, 6:7] = jnp.ones((_BE, 1), _F32)
        ea8_ref[:, 7:8] = jnp.zeros((_BE, 1), _F32)

    return pl.pallas_call(
        body,
        grid=(_EP // _BE,),
        in_specs=[pl.BlockSpec((_BE, 6), lambda i: (i, 0))],
        out_specs=pl.BlockSpec((_BE, 8), lambda i: (i, 0)),
        out_shape=jax.ShapeDtypeStruct((_EP, 8), _F32),
        compiler_params=pltpu.CompilerParams(
            dimension_semantics=("arbitrary",),
        ),
    )(edge_attr)


def _tc_node_prep(h):
    def body(h_ref, h0p_ref, hh2_ref):
        i = pl.program_id(0)
        row = i * _BN + lax.broadcasted_iota(jnp.int32, (_BN, 1), 0)
        hp = jnp.concatenate([h_ref[...], jnp.zeros((_BN, 5), _F32)], axis=1)
        hp = jnp.where(row < _N, hp, 0.0)
        h0p_ref[...] = hp
        hh2_ref[0] = hp[:, :40]
        hh2_ref[1] = hp[:, 40:]

    return pl.pallas_call(
        body,
        grid=(_NP // _BN,),
        in_specs=[pl.BlockSpec((_BN, 75), lambda i: (i, 0))],
        out_specs=[
            pl.BlockSpec((_BN, _DH), lambda i: (i, 0)),
            pl.BlockSpec((_NC, _BN, 40), lambda i: (0, i, 0)),
        ],
        out_shape=[
            jax.ShapeDtypeStruct((_NP, _DH), _F32),
            jax.ShapeDtypeStruct((_NC, _NP, 40), _F32),
        ],
        compiler_params=pltpu.CompilerParams(
            dimension_semantics=("arbitrary",),
        ),
    )(h)


def _sc_edge_stats(dst_p, ea8, zeros8):
    def body(dst_hbm, ea_hbm, z_hbm, out_hbm, acc, idx_v, ea_v):
        c = lax.axis_index("c")
        s = lax.axis_index("s")
        _zero_acc(z_hbm, acc, s)
        plsc.subcore_barrier()

        per_tile = _EP // (_NC * _NS)
        tile_base = (c * _NS + s) * per_tile
        nch = per_tile // _CH

        def ebody(j, carry):
            base = tile_base + j * _CH
            pltpu.sync_copy(dst_hbm.at[pl.ds(base, _CH)], idx_v)
            pltpu.sync_copy(ea_hbm.at[pl.ds(base, _CH)], ea_v)
            pltpu.sync_copy(ea_v, acc.at[idx_v], add=True)
            return carry

        lax.fori_loop(0, nch, ebody, 0)
        plsc.subcore_barrier()
        _copy_out(acc, out_hbm.at[c], s)

    return pl.kernel(
        body,
        out_type=jax.ShapeDtypeStruct((_NC, _NP, 8), _F32),
        mesh=_sc_mesh(),
        scratch_types=[
            pltpu.MemorySpace.VMEM_SHARED((_NP, 8), _F32),
            pltpu.VMEM((_CH,), jnp.int32),
            pltpu.VMEM((_CH, 8), _F32),
        ],
        compiler_params=pltpu.CompilerParams(use_tc_tiling_on_sc=False),
    )(dst_p, ea8, zeros8)


def _sc_gather_segsum(src_p, dst_p, hh2, zeros40):
    def body(src_hbm, dst_hbm, tab_hbm, z_hbm, out_hbm,
             acc, ss0, ss1, sd0, sd1, rows,
             gs0, gs1, is0, is1, ds0, ds1):
        c = lax.axis_index("c")
        s = lax.axis_index("s")
        gsem = [gs0, gs1]
        isem = [is0, is1]
        dsem = [ds0, ds1]
        ssidx = [ss0, ss1]
        sdidx = [sd0, sd1]
        _zero_acc(z_hbm, acc, s)
        plsc.subcore_barrier()

        per_tile = _EP // _NS
        nch = per_tile // _EC
        ebase = s * per_tile

        def sidx_start(j, b):
            pltpu.make_async_copy(src_hbm.at[pl.ds(ebase + j * _EC, _EC)],
                                  ssidx[b], isem[b]).start()

        def sidx_wait(j, b):
            pltpu.make_async_copy(src_hbm.at[pl.ds(ebase + j * _EC, _EC)],
                                  ssidx[b], isem[b]).wait()

        def didx_start(j, b):
            pltpu.make_async_copy(dst_hbm.at[pl.ds(ebase + j * _EC, _EC)],
                                  sdidx[b], dsem[b]).start()

        def didx_wait(j, b):
            pltpu.make_async_copy(dst_hbm.at[pl.ds(ebase + j * _EC, _EC)],
                                  sdidx[b], dsem[b]).wait()

        def gather_start(b):
            pltpu.make_async_copy(tab_hbm.at[c].at[ssidx[b]], rows.at[b],
                                  gsem[b]).start()

        def gather_wait(b):
            pltpu.make_async_copy(tab_hbm.at[c].at[ssidx[b]], rows.at[b],
                                  gsem[b]).wait()

        for b in (0, 1):
            sidx_start(b, b)
            didx_start(b, b)
        for b in (0, 1):
            sidx_wait(b, b)
            gather_start(b)

        def mbody(jj, carry):
            for b in range(2):
                j = 2 * jj + b
                gather_wait(b)
                didx_wait(j, b)
                pltpu.sync_copy(rows.at[b], acc.at[sdidx[b]], add=True)

                @pl.when(j + 2 < nch)
                def _():
                    didx_start(j + 2, b)
                    sidx_start(j + 2, b)
                    sidx_wait(j + 2, b)
                    gather_start(b)

            return carry

        lax.fori_loop(0, nch // 2, mbody, 0)
        plsc.subcore_barrier()
        _copy_out(acc, out_hbm.at[c], s)

    return pl.kernel(
        body,
        out_type=jax.ShapeDtypeStruct((_NC, _NP, 40), _F32),
        mesh=_sc_mesh(),
        scratch_types=[
            pltpu.MemorySpace.VMEM_SHARED((_NP, 40), _F32),
            pltpu.VMEM((_EC,), jnp.int32),
            pltpu.VMEM((_EC,), jnp.int32),
            pltpu.VMEM((_EC,), jnp.int32),
            pltpu.VMEM((_EC,), jnp.int32),
            pltpu.VMEM((2, _EC, 40), _F32),
            pltpu.SemaphoreType.DMA,
            pltpu.SemaphoreType.DMA,
            pltpu.SemaphoreType.DMA,
            pltpu.SemaphoreType.DMA,
            pltpu.SemaphoreType.DMA,
            pltpu.SemaphoreType.DMA,
        ],
        compiler_params=pltpu.CompilerParams(use_tc_tiling_on_sc=False),
    )(src_p, dst_p, hh2, zeros40)


def _tc_round(hh2, g2, s7p, At, Wt, Mt, bb):
    def body(hh_ref, g_ref, s_ref, At_ref, Wt_ref, Mt_ref, bb_ref, out_ref):
        hh = jnp.concatenate([hh_ref[0], hh_ref[1]], axis=1)
        g = jnp.concatenate([g_ref[0], g_ref[1]], axis=1)
        sv = s_ref[0] + s_ref[1]
        t = (jnp.dot(hh, At_ref[...], preferred_element_type=_F32)
             + jnp.dot(g, Wt_ref[...], preferred_element_type=_F32)
             + jnp.dot(sv, Mt_ref[...], preferred_element_type=_F32)
             + bb_ref[...])
        r = jnp.maximum(t, 0.0)
        out_ref[0] = r[:, :40]
        out_ref[1] = r[:, 40:]

    return pl.pallas_call(
        body,
        grid=(_NP // _BN,),
        in_specs=[
            pl.BlockSpec((_NC, _BN, 40), lambda i: (0, i, 0)),
            pl.BlockSpec((_NC, _BN, 40), lambda i: (0, i, 0)),
            pl.BlockSpec((_NC, _BN, 8), lambda i: (0, i, 0)),
            pl.BlockSpec((_DH, _DH), lambda i: (0, 0)),
            pl.BlockSpec((_DH, _DH), lambda i: (0, 0)),
            pl.BlockSpec((8, _DH), lambda i: (0, 0)),
            pl.BlockSpec((1, _DH), lambda i: (0, 0)),
        ],
        out_specs=pl.BlockSpec((_NC, _BN, 40), lambda i: (0, i, 0)),
        out_shape=jax.ShapeDtypeStruct((_NC, _NP, 40), _F32),
        compiler_params=pltpu.CompilerParams(
            dimension_semantics=("arbitrary",),
        ),
    )(hh2, g2, s7p, At, Wt, Mt, bb)


def _tc_readout(hh2, h0p, Pt, Qt, Rb, out_wt, out_b2):
    def body(hh_ref, h0_ref, Pt_ref, Qt_ref, Rb_ref, ow_ref, ob_ref,
             out_ref, acc_ref):
        i = pl.program_id(0)

        @pl.when(i == 0)
        def _():
            acc_ref[...] = jnp.zeros_like(acc_ref)

        hh = jnp.concatenate([hh_ref[0], hh_ref[1]], axis=1)
        reads = (jnp.dot(hh, Pt_ref[...], preferred_element_type=_F32)
                 + jnp.dot(h0_ref[...], Qt_ref[...], preferred_element_type=_F32)
                 + Rb_ref[...])
        reads = jnp.maximum(reads, 0.0)
        gid = i * _BN + lax.broadcasted_iota(jnp.int32, (_BN, 1), 0)
        reads = jnp.where(gid < _N, reads, 0.0)
        acc_ref[...] += jnp.sum(reads.reshape(_BN // 8, 8, 128), axis=0)

        @pl.when(i == _NP // _BN - 1)
        def _():
            tot = jnp.sum(acc_ref[...], axis=0, keepdims=True)
            v = jnp.tanh(tot)
            out_ref[...] = jnp.dot(v, ow_ref[...],
                                   preferred_element_type=_F32) + ob_ref[...]

    return pl.pallas_call(
        body,
        grid=(_NP // _BN,),
        in_specs=[
            pl.BlockSpec((_NC, _BN, 40), lambda i: (0, i, 0)),
            pl.BlockSpec((_BN, _DH), lambda i: (i, 0)),
            pl.BlockSpec((_DH, 128), lambda i: (0, 0)),
            pl.BlockSpec((_DH, 128), lambda i: (0, 0)),
            pl.BlockSpec((1, 128), lambda i: (0, 0)),
            pl.BlockSpec((128, 1), lambda i: (0, 0)),
            pl.BlockSpec((1, 1), lambda i: (0, 0)),
        ],
        out_specs=pl.BlockSpec((1, 1), lambda i: (0, 0)),
        out_shape=jax.ShapeDtypeStruct((1, 1), _F32),
        scratch_shapes=[pltpu.VMEM((8, 128), _F32)],
        compiler_params=pltpu.CompilerParams(
            dimension_semantics=("arbitrary",),
        ),
    )(hh2, h0p, Pt, Qt, Rb, out_wt, out_b2)


def kernel(h, edge_index, edge_attr, R_w, R_b, E_w, E_b,
           U0_w, U0_b, U1_w, U1_b, U2_w, U2_b,
           V0_w, V0_b, V1_w, V1_b, V2_w, V2_b,
           out_w, out_b):
    src_p, dst_p = _tc_idx_prep(edge_index)
    h0p, hh2 = _tc_node_prep(h)
    ea8 = _tc_ea_prep(edge_attr)

    zeros8 = jnp.zeros((_CH, 8), _F32)
    zeros40 = jnp.zeros((_CH, 40), _F32)

    Uw = [U0_w, U1_w, U2_w]
    Ub = [U0_b, U1_b, U2_b]
    Vw = [V0_w, V1_w, V2_w]
    Vb = [V0_b, V1_b, V2_b]
    Ats, Wts, Mts, bbs = [], [], [], []
    for k in range(3):
        A = Uw[k][:, :75]
        B = Uw[k][:, 75:150]
        C = Uw[k][:, 150:156]
        W = B @ Vw[k]
        At = jnp.zeros((_DH, _DH), _F32).at[:75, :75].set(A.T)
        Wt = jnp.zeros((_DH, _DH), _F32).at[:75, :75].set(W.T)
        m6 = C @ E_w
        v7 = C @ E_b + B @ Vb[k]
        Mt = jnp.zeros((8, _DH), _F32)
        Mt = Mt.at[:6, :75].set(m6.T).at[6, :75].set(v7)
        bb = jnp.zeros((1, _DH), _F32).at[0, :75].set(Ub[k])
        Ats.append(At); Wts.append(Wt); Mts.append(Mt); bbs.append(bb)

    Pt = jnp.zeros((_DH, 128), _F32).at[:75, :].set(R_w[:, :75].T)
    Qt = jnp.zeros((_DH, 128), _F32).at[:75, :].set(R_w[:, 75:].T)
    Rb2 = R_b.reshape(1, 128)
    out_wt = out_w.reshape(1, 128).T
    out_b2 = out_b.reshape(1, 1)

    g2 = _sc_gather_segsum(src_p, dst_p, hh2, zeros40)
    s7p = _sc_edge_stats(dst_p, ea8, zeros8)

    for k in range(3):
        hh2 = _tc_round(hh2, g2, s7p, Ats[k], Wts[k], Mts[k], bbs[k])
        if k < 2:
            g2 = _sc_gather_segsum(src_p, dst_p, hh2, zeros40)

    return _tc_readout(hh2, h0p, Pt, Qt, Rb2, out_wt, out_b2)


# speedup vs baseline: 4.6013x; 1.2785x over previous
import jax
import jax.numpy as jnp
from jax import lax
from jax.experimental import pallas as pl
from jax.experimental.pallas import tpu as pltpu
from jax.experimental.pallas import tpu_sc as plsc

_N = 50000
_E = 800000
_DH = 80
_CH = 128
_EC = 64
_NP = 50176
_EP = 802816
_NC = 2
_NS = 16
_ROWCHUNKS = _NP // _CH
_BN = 1024
_BE = 4096
_F32 = jnp.float32


def _sc_mesh():
    return plsc.VectorSubcoreMesh(core_axis_name="c", subcore_axis_name="s")


def _zero_acc(zv_hbm, acc, s):
    nz = (_ROWCHUNKS + _NS - 1) // _NS

    def zbody(j, carry):
        ch = s + j * _NS

        @pl.when(ch < _ROWCHUNKS)
        def _():
            pltpu.sync_copy(zv_hbm, acc.at[pl.ds(ch * _CH, _CH)])

        return carry

    lax.fori_loop(0, nz, zbody, 0)


def _copy_out(acc, out_view, s):
    nz = (_ROWCHUNKS + _NS - 1) // _NS

    def obody(j, carry):
        ch = s + j * _NS

        @pl.when(ch < _ROWCHUNKS)
        def _():
            pltpu.sync_copy(acc.at[pl.ds(ch * _CH, _CH)],
                            out_view.at[pl.ds(ch * _CH, _CH)])

        return carry

    lax.fori_loop(0, nz, obody, 0)


def _tc_ea_prep(edge_attr):
    def body(ea_ref, ea8_ref):
        ea8_ref[:, 0:6] = ea_ref[...]
        ea8_ref[:, 6:7] = jnp.ones((_BE, 1), _F32)
        ea8_ref[:, 7:8] = jnp.zeros((_BE, 1), _F32)

    return pl.pallas_call(
        body,
        grid=((_E + _BE - 1) // _BE,),
        in_specs=[pl.BlockSpec((_BE, 6), lambda i: (i, 0))],
        out_specs=pl.BlockSpec((_BE, 8), lambda i: (i, 0)),
        out_shape=jax.ShapeDtypeStruct((_E, 8), _F32),
        compiler_params=pltpu.CompilerParams(
            dimension_semantics=("arbitrary",),
        ),
    )(edge_attr)


def _tc_node_prep(h):
    def body(h_ref, h0p_ref, hh2_ref):
        i = pl.program_id(0)
        row = i * _BN + lax.broadcasted_iota(jnp.int32, (_BN, 1), 0)
        hp = jnp.concatenate([h_ref[...], jnp.zeros((_BN, 5), _F32)], axis=1)
        hp = jnp.where(row < _N, hp, 0.0)
        h0p_ref[...] = hp
        hh2_ref[0] = hp[:, :40]
        hh2_ref[1] = hp[:, 40:]

    return pl.pallas_call(
        body,
        grid=(_NP // _BN,),
        in_specs=[pl.BlockSpec((_BN, 75), lambda i: (i, 0))],
        out_specs=[
            pl.BlockSpec((_BN, _DH), lambda i: (i, 0)),
            pl.BlockSpec((_NC, _BN, 40), lambda i: (0, i, 0)),
        ],
        out_shape=[
            jax.ShapeDtypeStruct((_NP, _DH), _F32),
            jax.ShapeDtypeStruct((_NC, _NP, 40), _F32),
        ],
        compiler_params=pltpu.CompilerParams(
            dimension_semantics=("arbitrary",),
        ),
    )(h)


def _sc_edge_stats(edge_index, ea8, zeros8):
    def body(ei_hbm, ea_hbm, z_hbm, out_hbm, acc, idx_v, ea_v):
        c = lax.axis_index("c")
        s = lax.axis_index("s")
        _zero_acc(z_hbm, acc, s)
        plsc.subcore_barrier()

        nch_sc = (_E // _NC) // _CH
        nloop = (nch_sc + _NS - 1) // _NS

        def ebody(j, carry):
            ch = s + j * _NS

            @pl.when(ch < nch_sc)
            def _():
                base = c * (_E // _NC) + ch * _CH
                pltpu.sync_copy(ei_hbm.at[1, pl.ds(base, _CH)], idx_v)
                pltpu.sync_copy(ea_hbm.at[pl.ds(base, _CH)], ea_v)
                pltpu.sync_copy(ea_v, acc.at[idx_v], add=True)

            return carry

        lax.fori_loop(0, nloop, ebody, 0)
        plsc.subcore_barrier()
        _copy_out(acc, out_hbm.at[c], s)

    return pl.kernel(
        body,
        out_type=jax.ShapeDtypeStruct((_NC, _NP, 8), _F32),
        mesh=_sc_mesh(),
        scratch_types=[
            pltpu.MemorySpace.VMEM_SHARED((_NP, 8), _F32),
            pltpu.VMEM((_CH,), jnp.int32),
            pltpu.VMEM((_CH, 8), _F32),
        ],
        compiler_params=pltpu.CompilerParams(use_tc_tiling_on_sc=False),
    )(edge_index, ea8, zeros8)


def _sc_gather_segsum(edge_index, hh2, zeros40):
    def body(ei_hbm, tab_hbm, z_hbm, out_hbm,
             acc, ss0, ss1, sd0, sd1, rows,
             gs0, gs1, is0, is1, ds0, ds1):
        c = lax.axis_index("c")
        s = lax.axis_index("s")
        gsem = [gs0, gs1]
        isem = [is0, is1]
        dsem = [ds0, ds1]
        ssidx = [ss0, ss1]
        sdidx = [sd0, sd1]
        _zero_acc(z_hbm, acc, s)
        plsc.subcore_barrier()

        nch = _E // _EC
        nloop = (nch + _NS - 1) // _NS

        def sidx_start(j, b):
            pltpu.make_async_copy(ei_hbm.at[0, pl.ds((s + j * _NS) * _EC, _EC)],
                                  ssidx[b], isem[b]).start()

        def sidx_wait(j, b):
            pltpu.make_async_copy(ei_hbm.at[0, pl.ds((s + j * _NS) * _EC, _EC)],
                                  ssidx[b], isem[b]).wait()

        def didx_start(j, b):
            pltpu.make_async_copy(ei_hbm.at[1, pl.ds((s + j * _NS) * _EC, _EC)],
                                  sdidx[b], dsem[b]).start()

        def didx_wait(j, b):
            pltpu.make_async_copy(ei_hbm.at[1, pl.ds((s + j * _NS) * _EC, _EC)],
                                  sdidx[b], dsem[b]).wait()

        def gather_start(b):
            pltpu.make_async_copy(tab_hbm.at[c].at[ssidx[b]], rows.at[b],
                                  gsem[b]).start()

        def gather_wait(b):
            pltpu.make_async_copy(tab_hbm.at[c].at[ssidx[b]], rows.at[b],
                                  gsem[b]).wait()

        for b in (0, 1):
            sidx_start(b, b)
            didx_start(b, b)
        for b in (0, 1):
            sidx_wait(b, b)
            gather_start(b)

        def mbody(jj, carry):
            for b in range(2):
                j = 2 * jj + b

                @pl.when(s + j * _NS < nch)
                def _():
                    gather_wait(b)
                    didx_wait(j, b)
                    pltpu.sync_copy(rows.at[b], acc.at[sdidx[b]], add=True)

                    @pl.when(s + (j + 2) * _NS < nch)
                    def _():
                        didx_start(j + 2, b)
                        sidx_start(j + 2, b)
                        sidx_wait(j + 2, b)
                        gather_start(b)

            return carry

        lax.fori_loop(0, (nloop + 1) // 2, mbody, 0)
        plsc.subcore_barrier()
        _copy_out(acc, out_hbm.at[c], s)

    return pl.kernel(
        body,
        out_type=jax.ShapeDtypeStruct((_NC, _NP, 40), _F32),
        mesh=_sc_mesh(),
        scratch_types=[
            pltpu.MemorySpace.VMEM_SHARED((_NP, 40), _F32),
            pltpu.VMEM((_EC,), jnp.int32),
            pltpu.VMEM((_EC,), jnp.int32),
            pltpu.VMEM((_EC,), jnp.int32),
            pltpu.VMEM((_EC,), jnp.int32),
            pltpu.VMEM((2, _EC, 40), _F32),
            pltpu.SemaphoreType.DMA,
            pltpu.SemaphoreType.DMA,
            pltpu.SemaphoreType.DMA,
            pltpu.SemaphoreType.DMA,
            pltpu.SemaphoreType.DMA,
            pltpu.SemaphoreType.DMA,
        ],
        compiler_params=pltpu.CompilerParams(use_tc_tiling_on_sc=False),
    )(edge_index, hh2, zeros40)


def _tc_round(hh2, g2, s7p, At, Wt, Mt, bb):
    def body(hh_ref, g_ref, s_ref, At_ref, Wt_ref, Mt_ref, bb_ref, out_ref):
        hh = jnp.concatenate([hh_ref[0], hh_ref[1]], axis=1)
        g = jnp.concatenate([g_ref[0], g_ref[1]], axis=1)
        sv = s_ref[0] + s_ref[1]
        t = (jnp.dot(hh, At_ref[...], preferred_element_type=_F32)
             + jnp.dot(g, Wt_ref[...], preferred_element_type=_F32)
             + jnp.dot(sv, Mt_ref[...], preferred_element_type=_F32)
             + bb_ref[...])
        r = jnp.maximum(t, 0.0)
        out_ref[0] = r[:, :40]
        out_ref[1] = r[:, 40:]

    return pl.pallas_call(
        body,
        grid=(_NP // _BN,),
        in_specs=[
            pl.BlockSpec((_NC, _BN, 40), lambda i: (0, i, 0)),
            pl.BlockSpec((_NC, _BN, 40), lambda i: (0, i, 0)),
            pl.BlockSpec((_NC, _BN, 8), lambda i: (0, i, 0)),
            pl.BlockSpec((_DH, _DH), lambda i: (0, 0)),
            pl.BlockSpec((_DH, _DH), lambda i: (0, 0)),
            pl.BlockSpec((8, _DH), lambda i: (0, 0)),
            pl.BlockSpec((1, _DH), lambda i: (0, 0)),
        ],
        out_specs=pl.BlockSpec((_NC, _BN, 40), lambda i: (0, i, 0)),
        out_shape=jax.ShapeDtypeStruct((_NC, _NP, 40), _F32),
        compiler_params=pltpu.CompilerParams(
            dimension_semantics=("arbitrary",),
        ),
    )(hh2, g2, s7p, At, Wt, Mt, bb)


def _tc_readout(hh2, h0p, Pt, Qt, Rb, out_wt, out_b2):
    def body(hh_ref, h0_ref, Pt_ref, Qt_ref, Rb_ref, ow_ref, ob_ref,
             out_ref, acc_ref):
        i = pl.program_id(0)

        @pl.when(i == 0)
        def _():
            acc_ref[...] = jnp.zeros_like(acc_ref)

        hh = jnp.concatenate([hh_ref[0], hh_ref[1]], axis=1)
        reads = (jnp.dot(hh, Pt_ref[...], preferred_element_type=_F32)
                 + jnp.dot(h0_ref[...], Qt_ref[...], preferred_element_type=_F32)
                 + Rb_ref[...])
        reads = jnp.maximum(reads, 0.0)
        gid = i * _BN + lax.broadcasted_iota(jnp.int32, (_BN, 1), 0)
        reads = jnp.where(gid < _N, reads, 0.0)
        acc_ref[...] += jnp.sum(reads.reshape(_BN // 8, 8, 128), axis=0)

        @pl.when(i == _NP // _BN - 1)
        def _():
            tot = jnp.sum(acc_ref[...], axis=0, keepdims=True)
            v = jnp.tanh(tot)
            out_ref[...] = jnp.dot(v, ow_ref[...],
                                   preferred_element_type=_F32) + ob_ref[...]

    return pl.pallas_call(
        body,
        grid=(_NP // _BN,),
        in_specs=[
            pl.BlockSpec((_NC, _BN, 40), lambda i: (0, i, 0)),
            pl.BlockSpec((_BN, _DH), lambda i: (i, 0)),
            pl.BlockSpec((_DH, 128), lambda i: (0, 0)),
            pl.BlockSpec((_DH, 128), lambda i: (0, 0)),
            pl.BlockSpec((1, 128), lambda i: (0, 0)),
            pl.BlockSpec((128, 1), lambda i: (0, 0)),
            pl.BlockSpec((1, 1), lambda i: (0, 0)),
        ],
        out_specs=pl.BlockSpec((1, 1), lambda i: (0, 0)),
        out_shape=jax.ShapeDtypeStruct((1, 1), _F32),
        scratch_shapes=[pltpu.VMEM((8, 128), _F32)],
        compiler_params=pltpu.CompilerParams(
            dimension_semantics=("arbitrary",),
        ),
    )(hh2, h0p, Pt, Qt, Rb, out_wt, out_b2)


def kernel(h, edge_index, edge_attr, R_w, R_b, E_w, E_b,
           U0_w, U0_b, U1_w, U1_b, U2_w, U2_b,
           V0_w, V0_b, V1_w, V1_b, V2_w, V2_b,
           out_w, out_b):
    h0p, hh2 = _tc_node_prep(h)
    ea8 = _tc_ea_prep(edge_attr)

    zeros8 = jnp.zeros((_CH, 8), _F32)
    zeros40 = jnp.zeros((_CH, 40), _F32)

    Uw = [U0_w, U1_w, U2_w]
    Ub = [U0_b, U1_b, U2_b]
    Vw = [V0_w, V1_w, V2_w]
    Vb = [V0_b, V1_b, V2_b]
    Ats, Wts, Mts, bbs = [], [], [], []
    for k in range(3):
        A = Uw[k][:, :75]
        B = Uw[k][:, 75:150]
        C = Uw[k][:, 150:156]
        W = B @ Vw[k]
        At = jnp.zeros((_DH, _DH), _F32).at[:75, :75].set(A.T)
        Wt = jnp.zeros((_DH, _DH), _F32).at[:75, :75].set(W.T)
        m6 = C @ E_w
        v7 = C @ E_b + B @ Vb[k]
        Mt = jnp.zeros((8, _DH), _F32)
        Mt = Mt.at[:6, :75].set(m6.T).at[6, :75].set(v7)
        bb = jnp.zeros((1, _DH), _F32).at[0, :75].set(Ub[k])
        Ats.append(At); Wts.append(Wt); Mts.append(Mt); bbs.append(bb)

    Pt = jnp.zeros((_DH, 128), _F32).at[:75, :].set(R_w[:, :75].T)
    Qt = jnp.zeros((_DH, 128), _F32).at[:75, :].set(R_w[:, 75:].T)
    Rb2 = R_b.reshape(1, 128)
    out_wt = out_w.reshape(1, 128).T
    out_b2 = out_b.reshape(1, 1)

    g2 = _sc_gather_segsum(edge_index, hh2, zeros40)
    s7p = _sc_edge_stats(edge_index, ea8, zeros8)

    for k in range(3):
        hh2 = _tc_round(hh2, g2, s7p, Ats[k], Wts[k], Mts[k], bbs[k])
        if k < 2:
            g2 = _sc_gather_segsum(edge_index, hh2, zeros40)

    return _tc_readout(hh2, h0p, Pt, Qt, Rb2, out_wt, out_b2)
